# Initial kernel scaffold; baseline (speedup 1.0000x reference)
import jax
import jax.numpy as jnp
from jax.experimental import pallas as pl


def kernel(source_sentences, positions, emb_table, pos_table, W, b):
    raise NotImplementedError("write your pallas kernel here")


# speedup vs baseline: 1.4563x; 1.4563x over previous
import functools

import jax
import jax.numpy as jnp
from jax import lax
from jax.experimental import pallas as pl
from jax.experimental.pallas import tpu as pltpu
from jax.experimental.pallas import tpu_sc as plsc

VOCAB = 100000
MAX_LEN = 50
EMB = 128
HID = 256
B, L = 1024, 50
T = B * L
NW = 32
TPW = T // NW
CH = 80
NCH = TPW // CH


def _sc_gather(src_flat, pos_flat, emb_table, pos_table):
    mesh = plsc.VectorSubcoreMesh(core_axis_name="c", subcore_axis_name="s")

    @functools.partial(
        pl.kernel,
        out_type=jax.ShapeDtypeStruct((T, 2 * EMB), jnp.float32),
        mesh=mesh,
        scratch_types=[
            pltpu.VMEM((CH,), jnp.int32),
            pltpu.VMEM((CH,), jnp.int32),
            pltpu.VMEM((CH, EMB), jnp.float32),
            pltpu.VMEM((CH, EMB), jnp.float32),
            pltpu.SemaphoreType.DMA,
            pltpu.SemaphoreType.DMA,
        ],
    )
    def k(src_hbm, pos_hbm, emb_hbm, ptab_hbm, cat_hbm,
          idxw, idxp, wbuf, pbuf, semw, semp):
        wid = lax.axis_index("s") * 2 + lax.axis_index("c")
        base = wid * TPW

        def body(i, carry):
            tok = base + i * CH
            pltpu.sync_copy(src_hbm.at[pl.ds(tok, CH)], idxw)
            pltpu.sync_copy(pos_hbm.at[pl.ds(tok, CH)], idxp)
            cw = pltpu.async_copy(emb_hbm.at[idxw], wbuf, semw)
            cp = pltpu.async_copy(ptab_hbm.at[idxp], pbuf, semp)
            cw.wait()
            cp.wait()
            pltpu.sync_copy(wbuf, cat_hbm.at[pl.ds(tok, CH), pl.ds(0, EMB)])
            pltpu.sync_copy(pbuf, cat_hbm.at[pl.ds(tok, CH), pl.ds(EMB, EMB)])
            return carry

        lax.fori_loop(0, NCH, body, 0)

    return k(src_flat, pos_flat, emb_table, pos_table)


def _tc_head(cat3, W, b2):
    BLK = 128

    def body(cat_ref, w_ref, b_ref, out_ref):
        avg = jnp.mean(cat_ref[...], axis=1)
        out_ref[...] = lax.dot_general(
            avg, w_ref[...], (((1,), (1,)), ((), ())),
            preferred_element_type=jnp.float32) + b_ref[...]

    return pl.pallas_call(
        body,
        grid=(B // BLK,),
        in_specs=[
            pl.BlockSpec((BLK, L, 2 * EMB), lambda i: (i, 0, 0)),
            pl.BlockSpec((HID, 2 * EMB), lambda i: (0, 0)),
            pl.BlockSpec((1, HID), lambda i: (0, 0)),
        ],
        out_specs=pl.BlockSpec((BLK, HID), lambda i: (i, 0)),
        out_shape=jax.ShapeDtypeStruct((B, HID), jnp.float32),
    )(cat3, W, b2)


def kernel(source_sentences, positions, emb_table, pos_table, W, b):
    src_flat = source_sentences.reshape(T)
    pos_flat = positions.reshape(T)
    cat_flat = _sc_gather(src_flat, pos_flat, emb_table, pos_table)
    cat = cat_flat.reshape(B, L, 2 * EMB)
    hidden = _tc_head(cat, W, b.reshape(1, HID))
    h0 = hidden[None]
    return (cat, h0, h0)


# speedup vs baseline: 1.7106x; 1.1746x over previous
import functools

import jax
import jax.numpy as jnp
from jax import lax
from jax.experimental import pallas as pl
from jax.experimental.pallas import tpu as pltpu
from jax.experimental.pallas import tpu_sc as plsc

VOCAB = 100000
MAX_LEN = 50
EMB = 128
HID = 256
B, L = 1024, 50
T = B * L
NW = 32
TPW = T // NW
CH = 80
NCH = TPW // CH


def _sc_gather(src_flat, pos_flat, emb_table, pos_table):
    mesh = plsc.VectorSubcoreMesh(core_axis_name="c", subcore_axis_name="s")

    @functools.partial(
        pl.kernel,
        out_type=jax.ShapeDtypeStruct((T, 2 * EMB), jnp.float32),
        mesh=mesh,
        scratch_types=[
            pltpu.VMEM((CH,), jnp.int32),
            pltpu.VMEM((CH,), jnp.int32),
            pltpu.VMEM((CH, EMB), jnp.float32),
            pltpu.VMEM((CH, EMB), jnp.float32),
            pltpu.SemaphoreType.DMA,
            pltpu.SemaphoreType.DMA,
        ],
    )
    def k(src_hbm, pos_hbm, emb_hbm, ptab_hbm, cat_hbm,
          idxw, idxp, wbuf, pbuf, semw, semp):
        wid = lax.axis_index("s") * 2 + lax.axis_index("c")
        base = wid * TPW

        def body(i, carry):
            tok = base + i * CH
            pltpu.sync_copy(src_hbm.at[pl.ds(tok, CH)], idxw)
            pltpu.sync_copy(pos_hbm.at[pl.ds(tok, CH)], idxp)
            cw = pltpu.async_copy(emb_hbm.at[idxw], wbuf, semw)
            cp = pltpu.async_copy(ptab_hbm.at[idxp], pbuf, semp)
            cw.wait()
            cp.wait()
            pltpu.sync_copy(wbuf, cat_hbm.at[pl.ds(tok, CH), pl.ds(0, EMB)])
            pltpu.sync_copy(pbuf, cat_hbm.at[pl.ds(tok, CH), pl.ds(EMB, EMB)])
            return carry

        lax.fori_loop(0, NCH, body, 0)

    return k(src_flat, pos_flat, emb_table, pos_table)


def _tc_head(cat_flat, W, b2):
    BLK = 128

    def body(cat_ref, w_ref, b_ref, cat_out_ref, hid_ref):
        cat3 = cat_ref[...].reshape(BLK, L, 2 * EMB)
        cat_out_ref[...] = cat3
        avg = jnp.mean(cat3, axis=1)
        hid_ref[...] = lax.dot_general(
            avg, w_ref[...], (((1,), (1,)), ((), ())),
            preferred_element_type=jnp.float32) + b_ref[...]

    return pl.pallas_call(
        body,
        grid=(B // BLK,),
        in_specs=[
            pl.BlockSpec((BLK * L, 2 * EMB), lambda i: (i, 0)),
            pl.BlockSpec((HID, 2 * EMB), lambda i: (0, 0)),
            pl.BlockSpec((1, HID), lambda i: (0, 0)),
        ],
        out_specs=[
            pl.BlockSpec((BLK, L, 2 * EMB), lambda i: (i, 0, 0)),
            pl.BlockSpec((BLK, HID), lambda i: (i, 0)),
        ],
        out_shape=[
            jax.ShapeDtypeStruct((B, L, 2 * EMB), jnp.float32),
            jax.ShapeDtypeStruct((B, HID), jnp.float32),
        ],
    )(cat_flat, W, b2)


def kernel(source_sentences, positions, emb_table, pos_table, W, b):
    src_flat = source_sentences.reshape(T)
    pos_flat = positions.reshape(T)
    cat_flat = _sc_gather(src_flat, pos_flat, emb_table, pos_table)
    cat, hidden = _tc_head(cat_flat, W, b.reshape(1, HID))
    h0 = hidden[None]
    return (cat, h0, h0)


# speedup vs baseline: 1.7354x; 1.0145x over previous
import functools

import jax
import jax.numpy as jnp
from jax import lax
from jax.experimental import pallas as pl
from jax.experimental.pallas import tpu as pltpu
from jax.experimental.pallas import tpu_sc as plsc

VOCAB = 100000
MAX_LEN = 50
EMB = 128
HID = 256
B, L = 1024, 50
T = B * L
NW = 32
TPW = T // NW
CH = 80
NCH = TPW // CH


def _sc_gather(src_flat, pos_flat, emb_table, pos_table):
    mesh = plsc.VectorSubcoreMesh(core_axis_name="c", subcore_axis_name="s")

    @functools.partial(
        pl.kernel,
        out_type=jax.ShapeDtypeStruct((T, 2 * EMB), jnp.float32),
        mesh=mesh,
        scratch_types=[
            pltpu.VMEM((TPW,), jnp.int32),
            pltpu.VMEM((TPW,), jnp.int32),
            [pltpu.VMEM((CH, EMB), jnp.float32) for _ in range(2)],
            [pltpu.VMEM((CH, EMB), jnp.float32) for _ in range(2)],
            [pltpu.SemaphoreType.DMA for _ in range(4)],
            [pltpu.SemaphoreType.DMA for _ in range(4)],
        ],
    )
    def k(src_hbm, pos_hbm, emb_hbm, ptab_hbm, cat_hbm,
          idxw_all, idxp_all, wbufs, pbufs, gsems, wsems):
        wid = lax.axis_index("s") * 2 + lax.axis_index("c")
        base = wid * TPW
        pltpu.sync_copy(src_hbm.at[pl.ds(base, TPW)], idxw_all)
        pltpu.sync_copy(pos_hbm.at[pl.ds(base, TPW)], idxp_all)

        def start_gather(i):
            j = i % 2
            cw = pltpu.async_copy(
                emb_hbm.at[idxw_all.at[pl.ds(i * CH, CH)]], wbufs[j], gsems[j])
            cp = pltpu.async_copy(
                ptab_hbm.at[idxp_all.at[pl.ds(i * CH, CH)]], pbufs[j],
                gsems[2 + j])
            return cw, cp

        gath = {0: start_gather(0)}
        writes = {}
        for i in range(NCH):
            if i >= 1 and i + 1 < NCH:
                ww, wp = writes.pop(i - 1)
                ww.wait()
                wp.wait()
            if i + 1 < NCH:
                gath[i + 1] = start_gather(i + 1)
            cw, cp = gath.pop(i)
            cw.wait()
            cp.wait()
            tok = base + i * CH
            j = i % 2
            ww = pltpu.async_copy(
                wbufs[j], cat_hbm.at[pl.ds(tok, CH), pl.ds(0, EMB)], wsems[j])
            wp = pltpu.async_copy(
                pbufs[j], cat_hbm.at[pl.ds(tok, CH), pl.ds(EMB, EMB)],
                wsems[2 + j])
            writes[i] = (ww, wp)
        for ww, wp in writes.values():
            ww.wait()
            wp.wait()

    return k(src_flat, pos_flat, emb_table, pos_table)


def _tc_head(cat_flat, W, b2):
    BLK = 128

    def body(cat_ref, w_ref, b_ref, cat_out_ref, hid_ref):
        cat3 = cat_ref[...].reshape(BLK, L, 2 * EMB)
        cat_out_ref[...] = cat3
        avg = jnp.mean(cat3, axis=1)
        hid_ref[...] = lax.dot_general(
            avg, w_ref[...], (((1,), (1,)), ((), ())),
            preferred_element_type=jnp.float32) + b_ref[...]

    return pl.pallas_call(
        body,
        grid=(B // BLK,),
        in_specs=[
            pl.BlockSpec((BLK * L, 2 * EMB), lambda i: (i, 0)),
            pl.BlockSpec((HID, 2 * EMB), lambda i: (0, 0)),
            pl.BlockSpec((1, HID), lambda i: (0, 0)),
        ],
        out_specs=[
            pl.BlockSpec((BLK, L, 2 * EMB), lambda i: (i, 0, 0)),
            pl.BlockSpec((BLK, HID), lambda i: (i, 0)),
        ],
        out_shape=[
            jax.ShapeDtypeStruct((B, L, 2 * EMB), jnp.float32),
            jax.ShapeDtypeStruct((B, HID), jnp.float32),
        ],
    )(cat_flat, W, b2)


def kernel(source_sentences, positions, emb_table, pos_table, W, b):
    src_flat = source_sentences.reshape(T)
    pos_flat = positions.reshape(T)
    cat_flat = _sc_gather(src_flat, pos_flat, emb_table, pos_table)
    cat, hidden = _tc_head(cat_flat, W, b.reshape(1, HID))
    h0 = hidden[None]
    return (cat, h0, h0)
